# baseline (speedup 1.0000x reference)
import functools

import jax
import jax.numpy as jnp
from jax.experimental import pallas as pl
from jax.experimental.pallas import tpu as pltpu
from jax.experimental.pallas import tpu_sc as plsc

DIM = 64
L = 20
BBLK = 128
NEG = -9e15
ALPHA_SLOPE = 0.2


@functools.lru_cache(maxsize=None)
def _make_sc_gather(V, D, Btot):
    info = plsc.get_sparse_core_info()
    NC, NS = info.num_cores, info.num_subcores
    NW = NC * NS
    assert Btot % NW == 0 and (Btot // NW) % 8 == 0
    b_per_w = Btot // NW
    mesh = plsc.VectorSubcoreMesh(core_axis_name="c", subcore_axis_name="s")

    @functools.partial(
        pl.kernel,
        mesh=mesh,
        compiler_params=pltpu.CompilerParams(use_tc_tiling_on_sc=False),
        out_type=jax.ShapeDtypeStruct((Btot, D), jnp.float32),
        scratch_types=[
            pltpu.VMEM((b_per_w,), jnp.int32),
            pltpu.VMEM((b_per_w, D), jnp.float32),
            pltpu.SemaphoreType.DMA,
        ],
    )
    def gather_k(table_hbm, idx_hbm, out_hbm, idx_v, rows_v, sem):
        wid = jax.lax.axis_index("s") * NC + jax.lax.axis_index("c")
        base = wid * b_per_w
        pltpu.sync_copy(idx_hbm.at[pl.ds(base, b_per_w)], idx_v)
        pltpu.async_copy(table_hbm.at[idx_v], rows_v, sem).wait()
        pltpu.sync_copy(rows_v, out_hbm.at[pl.ds(base, b_per_w)])

    return gather_k


def _attn_body(h_ref, adj_ref, aw_ref, out_ref):
    H = h_ref[...].reshape(BBLK, L, DIM)
    aw = aw_ref[...]
    adj = adj_ref[...]

    Hb = H.astype(jnp.bfloat16)
    awb = aw.astype(jnp.bfloat16)
    alpha = jnp.full((BBLK, L, L), NEG, dtype=jnp.float32)
    for k in range(4):
        hw = Hb * awb[k, :][None, None, :]
        e = jax.lax.dot_general(hw, Hb, (((2,), (2,)), ((0,), (0,))),
                                preferred_element_type=jnp.float32)
        alpha = jnp.where(adj == k + 1, e, alpha)

    sel = (adj >= 1) & (adj <= 4)
    lrelu = jnp.where(alpha > 0, alpha, ALPHA_SLOPE * alpha)
    alpha = jnp.where(sel, lrelu, NEG)

    m = jnp.max(alpha, axis=2, keepdims=True)
    p = jnp.exp(alpha - m)
    s = jnp.sum(p, axis=2, keepdims=True)
    alpha = p / s
    out_ref[...] = jax.lax.dot_general(alpha, H, (((2,), (1,)), ((0,), (0,))),
                                       preferred_element_type=jnp.float32)


def _attention(h_flat, adj2, aw, B):
    return pl.pallas_call(
        _attn_body,
        grid=(B // BBLK,),
        in_specs=[
            pl.BlockSpec((BBLK * L, DIM), lambda i: (i, 0)),
            pl.BlockSpec((BBLK, L, L), lambda i: (i, 0, 0)),
            pl.BlockSpec((4, DIM), lambda i: (0, 0)),
        ],
        out_specs=pl.BlockSpec((BBLK, L, DIM), lambda i: (i, 0, 0)),
        out_shape=jax.ShapeDtypeStruct((B, L, DIM), jnp.float32),
        compiler_params=pltpu.CompilerParams(
            dimension_semantics=("parallel",)),
    )(h_flat, adj2, aw)


def kernel(inputs, adj, mask_item, item, embedding, a_0, a_1, a_2, a_3):
    B, Ls = inputs.shape
    V, D = embedding.shape
    idx = inputs.reshape(B * Ls)
    h_flat = _make_sc_gather(V, D, B * Ls)(embedding, idx)
    aw = jnp.concatenate([a_0, a_1, a_2, a_3], axis=1).T
    return _attention(h_flat, adj, aw, B)


# speedup vs baseline: 1.0246x; 1.0246x over previous
import functools

import jax
import jax.numpy as jnp
from jax.experimental import pallas as pl
from jax.experimental.pallas import tpu as pltpu
from jax.experimental.pallas import tpu_sc as plsc

DIM = 64
L = 20
BBLK = 256
NEG = -9e15
ALPHA_SLOPE = 0.2


@functools.lru_cache(maxsize=None)
def _make_sc_gather(V, D, Btot):
    info = plsc.get_sparse_core_info()
    NC, NS = info.num_cores, info.num_subcores
    NW = NC * NS
    assert Btot % NW == 0 and (Btot // NW) % 8 == 0
    b_per_w = Btot // NW
    mesh = plsc.VectorSubcoreMesh(core_axis_name="c", subcore_axis_name="s")

    @functools.partial(
        pl.kernel,
        mesh=mesh,
        compiler_params=pltpu.CompilerParams(use_tc_tiling_on_sc=False),
        out_type=jax.ShapeDtypeStruct((Btot, D), jnp.float32),
        scratch_types=[
            pltpu.VMEM((b_per_w,), jnp.int32),
            pltpu.VMEM((b_per_w, D), jnp.float32),
            pltpu.SemaphoreType.DMA,
        ],
    )
    def gather_k(table_hbm, idx_hbm, out_hbm, idx_v, rows_v, sem):
        wid = jax.lax.axis_index("s") * NC + jax.lax.axis_index("c")
        base = wid * b_per_w
        pltpu.sync_copy(idx_hbm.at[pl.ds(base, b_per_w)], idx_v)
        pltpu.async_copy(table_hbm.at[idx_v], rows_v, sem).wait()
        pltpu.sync_copy(rows_v, out_hbm.at[pl.ds(base, b_per_w)])

    return gather_k


def _attn_body(h_ref, adj_ref, aw_ref, out_ref):
    H = h_ref[...].reshape(BBLK, L, DIM)
    aw = aw_ref[...]
    adj = adj_ref[...].reshape(BBLK, L, L)

    Hb = H.astype(jnp.bfloat16)
    awb = aw.astype(jnp.bfloat16)
    alpha = jnp.full((BBLK, L, L), NEG, dtype=jnp.float32)
    for k in range(4):
        hw = Hb * awb[k, :][None, None, :]
        e = jax.lax.dot_general(hw, Hb, (((2,), (2,)), ((0,), (0,))),
                                preferred_element_type=jnp.float32)
        alpha = jnp.where(adj == k + 1, e, alpha)

    sel = (adj >= 1) & (adj <= 4)
    lrelu = jnp.where(alpha > 0, alpha, ALPHA_SLOPE * alpha)
    alpha = jnp.where(sel, lrelu, NEG)

    m = jnp.max(alpha, axis=2, keepdims=True)
    p = jnp.exp(alpha - m)
    s = jnp.sum(p, axis=2, keepdims=True)
    alpha = p / s
    out_ref[...] = jax.lax.dot_general(alpha, H, (((2,), (1,)), ((0,), (0,))),
                                       preferred_element_type=jnp.float32)


def _attention(h_flat, adj2, aw, B):
    return pl.pallas_call(
        _attn_body,
        grid=(B // BBLK,),
        in_specs=[
            pl.BlockSpec((BBLK * L, DIM), lambda i: (i, 0)),
            pl.BlockSpec((BBLK, L * L), lambda i: (i, 0)),
            pl.BlockSpec((4, DIM), lambda i: (0, 0)),
        ],
        out_specs=pl.BlockSpec((BBLK, L, DIM), lambda i: (i, 0, 0)),
        out_shape=jax.ShapeDtypeStruct((B, L, DIM), jnp.float32),
        compiler_params=pltpu.CompilerParams(
            dimension_semantics=("parallel",)),
    )(h_flat, adj2, aw)


def kernel(inputs, adj, mask_item, item, embedding, a_0, a_1, a_2, a_3):
    B, Ls = inputs.shape
    V, D = embedding.shape
    idx = inputs.reshape(B * Ls)
    h_flat = _make_sc_gather(V, D, B * Ls)(embedding, idx)
    aw = jnp.concatenate([a_0, a_1, a_2, a_3], axis=1).T
    return _attention(h_flat, adj.reshape(B, Ls * Ls), aw, B)


# speedup vs baseline: 1.0381x; 1.0132x over previous
import functools

import jax
import jax.numpy as jnp
from jax.experimental import pallas as pl
from jax.experimental.pallas import tpu as pltpu
from jax.experimental.pallas import tpu_sc as plsc

DIM = 64
L = 20
BBLK = 128
NEG = -9e15
ALPHA_SLOPE = 0.2


@functools.lru_cache(maxsize=None)
def _make_sc_gather(V, D, Btot):
    info = plsc.get_sparse_core_info()
    NC, NS = info.num_cores, info.num_subcores
    NW = NC * NS
    assert Btot % NW == 0 and (Btot // NW) % 8 == 0
    b_per_w = Btot // NW
    mesh = plsc.VectorSubcoreMesh(core_axis_name="c", subcore_axis_name="s")

    @functools.partial(
        pl.kernel,
        mesh=mesh,
        compiler_params=pltpu.CompilerParams(use_tc_tiling_on_sc=False),
        out_type=jax.ShapeDtypeStruct((Btot, D), jnp.float32),
        scratch_types=[
            pltpu.VMEM((b_per_w,), jnp.int32),
            pltpu.VMEM((b_per_w, D), jnp.float32),
            pltpu.SemaphoreType.DMA,
        ],
    )
    def gather_k(table_hbm, idx_hbm, out_hbm, idx_v, rows_v, sem):
        wid = jax.lax.axis_index("s") * NC + jax.lax.axis_index("c")
        base = wid * b_per_w
        pltpu.sync_copy(idx_hbm.at[pl.ds(base, b_per_w)], idx_v)
        pltpu.async_copy(table_hbm.at[idx_v], rows_v, sem).wait()
        pltpu.sync_copy(rows_v, out_hbm.at[pl.ds(base, b_per_w)])

    return gather_k


def _attn_body(h_ref, adj_ref, aw_ref, out_ref):
    H = h_ref[...].reshape(BBLK, L, DIM)
    aw = aw_ref[...]
    adj = adj_ref[...].reshape(BBLK, L, L)

    Hb = H.astype(jnp.bfloat16)
    awb = aw.astype(jnp.bfloat16)
    alpha = jnp.full((BBLK, L, L), NEG, dtype=jnp.float32)
    for k in range(4):
        hw = Hb * awb[k, :][None, None, :]
        e = jax.lax.dot_general(hw, Hb, (((2,), (2,)), ((0,), (0,))),
                                preferred_element_type=jnp.float32)
        alpha = jnp.where(adj == k + 1, e, alpha)

    alpha = jnp.where(alpha > 0, alpha, ALPHA_SLOPE * alpha)

    m = jnp.max(alpha, axis=2, keepdims=True)
    p = jnp.exp(alpha - m)
    s = jnp.sum(p, axis=2, keepdims=True)
    alpha = p / s
    out_ref[...] = jax.lax.dot_general(alpha, H, (((2,), (1,)), ((0,), (0,))),
                                       preferred_element_type=jnp.float32)


def _attention(h_flat, adj2, aw, B):
    return pl.pallas_call(
        _attn_body,
        grid=(B // BBLK,),
        in_specs=[
            pl.BlockSpec((BBLK * L, DIM), lambda i: (i, 0)),
            pl.BlockSpec((BBLK, L * L), lambda i: (i, 0)),
            pl.BlockSpec((4, DIM), lambda i: (0, 0)),
        ],
        out_specs=pl.BlockSpec((BBLK, L, DIM), lambda i: (i, 0, 0)),
        out_shape=jax.ShapeDtypeStruct((B, L, DIM), jnp.float32),
        compiler_params=pltpu.CompilerParams(
            dimension_semantics=("parallel",)),
    )(h_flat, adj2, aw)


def kernel(inputs, adj, mask_item, item, embedding, a_0, a_1, a_2, a_3):
    B, Ls = inputs.shape
    V, D = embedding.shape
    idx = inputs.reshape(B * Ls)
    h_flat = _make_sc_gather(V, D, B * Ls)(embedding, idx)
    aw = jnp.concatenate([a_0, a_1, a_2, a_3], axis=1).T
    return _attention(h_flat, adj.reshape(B, Ls * Ls), aw, B)
